# Initial kernel scaffold; baseline (speedup 1.0000x reference)
import jax
import jax.numpy as jnp
from jax.experimental import pallas as pl


def kernel(x, edge_index, W1, b1, W2, b2):
    raise NotImplementedError("write your pallas kernel here")


# speedup vs baseline: 3.7543x; 3.7543x over previous
import jax
import jax.numpy as jnp
from jax import lax
from jax.experimental import pallas as pl
from jax.experimental.pallas import tpu as pltpu
from jax.experimental.pallas import tpu_sc as plsc

N_NODES = 10000
D = 128
E = 320000
NC = 2
NS = 16
NW = NC * NS
EPW = E // NW
CHUNK = 128
NCHUNKS = 80
PAD = NCHUNKS * CHUNK - EPW
N_ACC = N_NODES + 8
RPW = 624
ZROWS = 16


def _sc_aggregate(x, src, dst):
    mesh = plsc.VectorSubcoreMesh(core_axis_name="c", subcore_axis_name="s")

    def body(x_hbm, src_hbm, dst_hbm, out_hbm, src_v, dst_v, gbuf, zbuf, accum,
             zsem):
        cid = lax.axis_index("c")
        sid = lax.axis_index("s")
        wid = sid * NC + cid

        @pl.loop(0, ZROWS)
        def _(r):
            @pl.loop(0, D // 16)
            def _(c):
                zbuf[r, pl.ds(c * 16, 16)] = jnp.zeros((16,), jnp.float32)

        row0 = sid * RPW
        descs = [pltpu.async_copy(zbuf, accum.at[pl.ds(row0 + i * ZROWS, ZROWS)],
                                  zsem)
                 for i in range(RPW // ZROWS)]

        @pl.when(sid == NS - 1)
        def _():
            pltpu.async_copy(zbuf, accum.at[pl.ds(row0 + RPW, ZROWS)], zsem).wait()

        for d in descs:
            d.wait()
        plsc.subcore_barrier()

        pltpu.sync_copy(src_hbm.at[wid], src_v)
        pltpu.sync_copy(dst_hbm.at[wid], dst_v)

        @pl.loop(0, NCHUNKS)
        def _(cnk):
            pltpu.sync_copy(x_hbm.at[src_v.at[cnk]], gbuf)
            pltpu.sync_copy(gbuf, accum.at[dst_v.at[cnk]], add=True)

        plsc.subcore_barrier()

        @pl.when(sid < NS - 1)
        def _():
            pltpu.sync_copy(accum.at[pl.ds(row0, RPW)],
                            out_hbm.at[cid, pl.ds(row0, RPW)])

        @pl.when(sid == NS - 1)
        def _():
            pltpu.sync_copy(accum.at[pl.ds(row0, RPW + ZROWS)],
                            out_hbm.at[cid, pl.ds(row0, RPW + ZROWS)])

    f = pl.kernel(
        body,
        out_type=jax.ShapeDtypeStruct((NC, N_NODES, D), jnp.float32),
        mesh=mesh,
        scratch_types=[
            pltpu.VMEM((NCHUNKS, CHUNK), jnp.int32),
            pltpu.VMEM((NCHUNKS, CHUNK), jnp.int32),
            pltpu.VMEM((CHUNK, D), jnp.float32),
            pltpu.VMEM((ZROWS, D), jnp.float32),
            pltpu.VMEM_SHARED((N_ACC, D), jnp.float32),
            pltpu.SemaphoreType.DMA,
        ],
    )
    return f(x, src, dst)


def _tc_mlp(x, p0, p1, W1, b1, W2, b2):
    BR = 2000
    dn = (((1,), (0,)), ((), ()))

    def body(x_ref, p0_ref, p1_ref, w1_ref, b1_ref, w2_ref, b2_ref, o_ref):
        h = x_ref[...] + p0_ref[...] + p1_ref[...]
        h1 = lax.dot_general(h, w1_ref[...], dn,
                             precision=lax.Precision.HIGHEST,
                             preferred_element_type=jnp.float32) + b1_ref[...]
        h1 = jnp.maximum(h1, 0.0)
        o_ref[...] = lax.dot_general(h1, w2_ref[...], dn,
                                     precision=lax.Precision.HIGHEST,
                                     preferred_element_type=jnp.float32) + b2_ref[...]

    return pl.pallas_call(
        body,
        grid=(N_NODES // BR,),
        in_specs=[
            pl.BlockSpec((BR, D), lambda i: (i, 0)),
            pl.BlockSpec((BR, D), lambda i: (i, 0)),
            pl.BlockSpec((BR, D), lambda i: (i, 0)),
            pl.BlockSpec((D, D), lambda i: (0, 0)),
            pl.BlockSpec((1, D), lambda i: (0, 0)),
            pl.BlockSpec((D, D), lambda i: (0, 0)),
            pl.BlockSpec((1, D), lambda i: (0, 0)),
        ],
        out_specs=pl.BlockSpec((BR, D), lambda i: (i, 0)),
        out_shape=jax.ShapeDtypeStruct((N_NODES, D), jnp.float32),
    )(x, p0, p1, W1, b1.reshape(1, D), W2, b2.reshape(1, D))


def kernel(x, edge_index, W1, b1, W2, b2):
    src_pad = jnp.zeros((NW, PAD), jnp.int32)
    dst_pad = jnp.full((NW, PAD), N_NODES, jnp.int32)
    src = jnp.concatenate([edge_index[0].reshape(NW, EPW), src_pad],
                          axis=1).reshape(NW, NCHUNKS, CHUNK)
    dst = jnp.concatenate([edge_index[1].reshape(NW, EPW), dst_pad],
                          axis=1).reshape(NW, NCHUNKS, CHUNK)
    p = _sc_aggregate(x, src, dst)
    return _tc_mlp(x, p[0], p[1], W1, b1, W2, b2)


# speedup vs baseline: 4.3153x; 1.1494x over previous
import jax
import jax.numpy as jnp
from jax import lax
from jax.experimental import pallas as pl
from jax.experimental.pallas import tpu as pltpu
from jax.experimental.pallas import tpu_sc as plsc

N_NODES = 10000
D = 128
E = 320000
NC = 2
NS = 16
NW = NC * NS
EPW = E // NW
CHUNK = 128
NCHUNKS = 80
PAD = NCHUNKS * CHUNK - EPW
N_ACC = N_NODES + 8
RPW = 624
ZROWS = 16
NBUF = 2
NG = NCHUNKS // NBUF


def _sc_aggregate(x, src, dst):
    mesh = plsc.VectorSubcoreMesh(core_axis_name="c", subcore_axis_name="s")

    def body(x_hbm, src_hbm, dst_hbm, out_hbm, src_v,
             gb0, gb1, db0, db1, zbuf, accum, zsem, isem, gsem, ssem, xsem):
        gbufs = (gb0, gb1)
        dbufs = (db0, db1)
        cid = lax.axis_index("c")
        sid = lax.axis_index("s")
        wid = sid * NC + cid

        isrc = pltpu.async_copy(src_hbm.at[wid], src_v, isem)

        @pl.loop(0, ZROWS)
        def _(r):
            @pl.loop(0, D // 16)
            def _(c):
                zbuf[r, pl.ds(c * 16, 16)] = jnp.zeros((16,), jnp.float32)

        row0 = sid * RPW
        descs = [pltpu.async_copy(zbuf, accum.at[pl.ds(row0 + i * ZROWS, ZROWS)],
                                  zsem)
                 for i in range(RPW // ZROWS)]

        @pl.when(sid == NS - 1)
        def _():
            pltpu.async_copy(zbuf, accum.at[pl.ds(row0 + RPW, ZROWS)], zsem).wait()

        for d in descs:
            d.wait()
        isrc.wait()
        plsc.subcore_barrier()

        for b in range(NBUF):
            pltpu.async_copy(dst_hbm.at[wid, b], dbufs[b], xsem.at[b])
            pltpu.async_copy(x_hbm.at[src_v.at[b]], gbufs[b], gsem.at[b])

        @pl.loop(0, NG)
        def _(g):
            c0 = g * NBUF
            for b in range(NBUF):
                c = c0 + b
                pltpu.make_async_copy(x_hbm.at[src_v.at[0]], gbufs[b],
                                      gsem.at[b]).wait()
                pltpu.make_async_copy(dst_hbm.at[wid, 0], dbufs[b],
                                      xsem.at[b]).wait()
                pltpu.async_copy(gbufs[b], accum.at[dbufs[b]], ssem.at[b],
                                 add=True)

                @pl.when(g < NG - 1)
                def _():
                    pltpu.make_async_copy(gbufs[b], accum.at[dbufs[b]],
                                          ssem.at[b]).wait()
                    pltpu.async_copy(dst_hbm.at[wid, c + NBUF], dbufs[b],
                                     xsem.at[b])
                    pltpu.async_copy(x_hbm.at[src_v.at[c + NBUF]], gbufs[b],
                                     gsem.at[b])

        for b in range(NBUF):
            pltpu.make_async_copy(gbufs[b], accum.at[dbufs[b]],
                                  ssem.at[b]).wait()
        plsc.subcore_barrier()

        @pl.when(sid < NS - 1)
        def _():
            pltpu.sync_copy(accum.at[pl.ds(row0, RPW)],
                            out_hbm.at[cid, pl.ds(row0, RPW)])

        @pl.when(sid == NS - 1)
        def _():
            pltpu.sync_copy(accum.at[pl.ds(row0, RPW + ZROWS)],
                            out_hbm.at[cid, pl.ds(row0, RPW + ZROWS)])

    f = pl.kernel(
        body,
        out_type=jax.ShapeDtypeStruct((NC, N_NODES, D), jnp.float32),
        mesh=mesh,
        scratch_types=[
            pltpu.VMEM((NCHUNKS, CHUNK), jnp.int32),
        ] + [pltpu.VMEM((CHUNK, D), jnp.float32) for _ in range(NBUF)] + [
            pltpu.VMEM((CHUNK,), jnp.int32) for _ in range(NBUF)] + [
            pltpu.VMEM((ZROWS, D), jnp.float32),
            pltpu.VMEM_SHARED((N_ACC, D), jnp.float32),
            pltpu.SemaphoreType.DMA,
            pltpu.SemaphoreType.DMA,
            pltpu.SemaphoreType.DMA((NBUF,)),
            pltpu.SemaphoreType.DMA((NBUF,)),
            pltpu.SemaphoreType.DMA((NBUF,)),
        ],
    )
    return f(x, src, dst)


def _tc_mlp(x, p0, p1, W1, b1, W2, b2):
    BR = 2000
    dn = (((1,), (0,)), ((), ()))

    def body(x_ref, p0_ref, p1_ref, w1_ref, b1_ref, w2_ref, b2_ref, o_ref):
        h = x_ref[...] + p0_ref[...] + p1_ref[...]
        h1 = lax.dot_general(h, w1_ref[...], dn,
                             precision=lax.Precision.HIGHEST,
                             preferred_element_type=jnp.float32) + b1_ref[...]
        h1 = jnp.maximum(h1, 0.0)
        o_ref[...] = lax.dot_general(h1, w2_ref[...], dn,
                                     precision=lax.Precision.HIGHEST,
                                     preferred_element_type=jnp.float32) + b2_ref[...]

    return pl.pallas_call(
        body,
        grid=(N_NODES // BR,),
        in_specs=[
            pl.BlockSpec((BR, D), lambda i: (i, 0)),
            pl.BlockSpec((BR, D), lambda i: (i, 0)),
            pl.BlockSpec((BR, D), lambda i: (i, 0)),
            pl.BlockSpec((D, D), lambda i: (0, 0)),
            pl.BlockSpec((1, D), lambda i: (0, 0)),
            pl.BlockSpec((D, D), lambda i: (0, 0)),
            pl.BlockSpec((1, D), lambda i: (0, 0)),
        ],
        out_specs=pl.BlockSpec((BR, D), lambda i: (i, 0)),
        out_shape=jax.ShapeDtypeStruct((N_NODES, D), jnp.float32),
    )(x, p0, p1, W1, b1.reshape(1, D), W2, b2.reshape(1, D))


def kernel(x, edge_index, W1, b1, W2, b2):
    src_pad = jnp.zeros((NW, PAD), jnp.int32)
    dst_pad = jnp.full((NW, PAD), N_NODES, jnp.int32)
    src = jnp.concatenate([edge_index[0].reshape(NW, EPW), src_pad],
                          axis=1).reshape(NW, NCHUNKS, CHUNK)
    dst = jnp.concatenate([edge_index[1].reshape(NW, EPW), dst_pad],
                          axis=1).reshape(NW, NCHUNKS, CHUNK)
    p = _sc_aggregate(x, src, dst)
    return _tc_mlp(x, p[0], p[1], W1, b1, W2, b2)
